# baseline (speedup 1.0000x reference)
import jax
import jax.numpy as jnp
from jax import lax
from jax.experimental import pallas as pl
from jax.experimental.pallas import tpu as pltpu
from jax.experimental.pallas import tpu_sc as plsc

_K = 100
_KP = 128
_C = 91
_N = 27300
_NW = 8
_NP = 27392
_SH = _NP // _NW
_SNV = _SH // 16
_QP = 304
_CAP = _SH + 16
_BIG = 1 << 28


def _sc_body(prob_hbm, cent_hbm, s_hbm, l_hbm, m_hbm,
             pv, cv, keys, gidx, hist, stk, sgi, lv, mv, mkf, mgf,
             skeys_sh, sgidx_sh):
    cid = lax.axis_index("c")
    sid = lax.axis_index("s")
    img = sid // _NW
    b = img * 2 + cid
    shard = sid % _NW
    is_lead = shard == 0

    pltpu.sync_copy(prob_hbm.at[pl.ds(b * _NP + shard * _SH, _SH)], pv)

    @pl.when(is_lead)
    def _():
        pltpu.sync_copy(cent_hbm.at[pl.ds(b * _QP * 4, _QP * 4)], cv)

    lane = lax.broadcasted_iota(jnp.int32, (16,), 0)
    ones = jnp.ones((16,), jnp.int32)
    zeros16 = jnp.zeros((16,), jnp.int32)
    big16 = jnp.full((16,), _BIG, jnp.int32)

    def clr(j, _):
        hist[pl.ds(j * 16, 16)] = zeros16
        return 0

    def suffix_scan(acc0, nbins):
        def cond(carry):
            bin_, acc, _ = carry
            return (acc < _K) & (bin_ > 0)

        def body(carry):
            bin_, acc, _ = carry
            bin2 = bin_ - 1
            tsum = jnp.sum(hist[pl.ds(bin2 * 16, 16)])
            return bin2, acc + tsum, acc

        bsel, acc, nab = lax.while_loop(
            cond, body, (jnp.int32(nbins), acc0, acc0))
        return acc, bsel, nab

    lax.fori_loop(0, 64, clr, 0)

    def ha(i, _):
        k1 = pv[pl.ds(i * 16, 16)]
        plsc.addupdate_scatter(hist, [(k1 >> 24) * 16 + lane], ones)
        return 0
    lax.fori_loop(0, _SNV, ha, 0)
    _, b1, nab1 = suffix_scan(jnp.int32(0), 64)

    lax.fori_loop(0, 256, clr, 0)

    def hb(i, _):
        k1 = pv[pl.ds(i * 16, 16)]
        plsc.addupdate_scatter(
            hist, [((k1 >> 16) & 0xFF) * 16 + lane], ones,
            mask=(k1 >> 24) == b1)
        return 0
    lax.fori_loop(0, _SNV, hb, 0)
    _, b2, _ = suffix_scan(nab1, 256)
    t16 = b1 * 256 + b2

    gbase = shard * _SH

    def cp(i, off):
        k1 = pv[pl.ds(i * 16, 16)]
        m = (k1 >> 16) >= t16
        mi = m.astype(jnp.int32)
        incl = plsc.cumsum(mi)
        dest = off + incl - mi
        plsc.store_scatter(keys, [dest], k1, mask=m)
        plsc.store_scatter(gidx, [dest], gbase + i * 16 + lane, mask=m)
        return off + incl[15]
    s_cnt = lax.fori_loop(0, _SNV, cp, jnp.int32(0))

    plsc.store_scatter(keys, [s_cnt + lane], zeros16)
    plsc.store_scatter(gidx, [s_cnt + lane], big16)
    nv = (s_cnt + 15) // 16

    for j in range(_KP // 16):
        sl = pl.ds(j * 16, 16)
        stk[sl] = zeros16
        sgi[sl] = big16

    lane0 = lane == 0
    neg16 = jnp.full((16,), -1, jnp.int32)

    def emit(k, mkey, midx):
        k16 = jnp.broadcast_to(k, (16,))
        plsc.store_scatter(stk, [k16], jnp.broadcast_to(mkey, (16,)),
                           mask=lane0)
        plsc.store_scatter(sgi, [k16], jnp.broadcast_to(midx, (16,)),
                           mask=lane0)

    def mn_sweep(mkey):
        def mn(j, vi):
            kv = keys[pl.ds(j * 16, 16)]
            return jnp.minimum(
                vi, jnp.where(kv == mkey, gidx[pl.ds(j * 16, 16)],
                              jnp.int32(_BIG)))
        vi = lax.fori_loop(0, nv, mn, big16)
        return jnp.min(vi)

    def mark_sweep(mkey, midx):
        def mark(j, _):
            kv = keys[pl.ds(j * 16, 16)]
            hit = (kv == mkey) & (gidx[pl.ds(j * 16, 16)] == midx)
            keys[pl.ds(j * 16, 16)] = jnp.where(hit, jnp.int32(-1), kv)
            return 0
        lax.fori_loop(0, nv, mark, 0)

    def build_summary(j, sm):
        mj = jnp.max(keys[pl.ds(j * 16, 16)])
        return jnp.where(lane == j, mj, sm)

    @pl.when(nv <= 16)
    def _():
        summary0 = lax.fori_loop(0, nv, build_summary, neg16)

        def sel(k, summary):
            mkey = jnp.max(summary)
            hits = summary == mkey
            nhit = plsc.all_reduce_population_count(hits)[0]
            jstar = plsc.all_reduce_ffs(hits)[0]

            def fast(_):
                sl = pl.ds(jstar * 16, 16)
                kv = keys[sl]
                gv = gidx[sl]
                midx = jnp.min(jnp.where(kv == mkey, gv, jnp.int32(_BIG)))
                kv2 = jnp.where((kv == mkey) & (gv == midx),
                                jnp.int32(-1), kv)
                keys[sl] = kv2
                sm2 = jnp.where(lane == jstar, jnp.max(kv2), summary)
                return midx, sm2

            def slow(_):
                midx = mn_sweep(mkey)
                mark_sweep(mkey, midx)
                sm2 = lax.fori_loop(0, nv, build_summary, neg16)
                return midx, sm2

            midx, sm2 = lax.cond(nhit == 1, fast, slow, 0)
            emit(k, mkey, midx)
            return sm2
        lax.fori_loop(0, _K, sel, summary0)

    @pl.when(nv > 16)
    def _():
        def sel(k, _):
            def mx(j, vm):
                return jnp.maximum(vm, keys[pl.ds(j * 16, 16)])
            vm = lax.fori_loop(0, nv, mx, neg16)
            mkey = jnp.max(vm)
            midx = mn_sweep(mkey)
            mark_sweep(mkey, midx)
            emit(k, mkey, midx)
            return 0
        lax.fori_loop(0, _K, sel, 0)

    slot = img * _NW + shard
    pltpu.sync_copy(stk, skeys_sh.at[pl.ds(slot * _KP, _KP)])
    pltpu.sync_copy(sgi, sgidx_sh.at[pl.ds(slot * _KP, _KP)])
    plsc.subcore_barrier()

    @pl.when(is_lead)
    def _():
        nwk = _NW * _KP
        pltpu.sync_copy(skeys_sh.at[pl.ds(img * nwk, nwk)],
                        mkf.at[pl.ds(0, nwk)])
        pltpu.sync_copy(sgidx_sh.at[pl.ds(img * nwk, nwk)],
                        mgf.at[pl.ds(0, nwk)])
        def padr(j, _):
            mkf[pl.ds(nwk + j * 16, 16)] = zeros16
            mgf[pl.ds(nwk + j * 16, 16)] = big16
            return 0
        lax.fori_loop(0, nwk // 16, padr, 0)

        lane0 = lane == 0

        def mrg(k, cvec):
            hk = plsc.load_gather(mkf, [lane * _KP + cvec])
            hg = plsc.load_gather(mgf, [lane * _KP + cvec])
            bk = jnp.max(hk)
            bg = jnp.min(jnp.where(hk == bk, hg, jnp.int32(_BIG)))
            win = (hk == bk) & (hg == bg)
            k16 = jnp.broadcast_to(k, (16,))
            plsc.store_scatter(stk, [k16], jnp.broadcast_to(bk, (16,)),
                               mask=lane0)
            plsc.store_scatter(sgi, [k16], jnp.broadcast_to(bg, (16,)),
                               mask=lane0)
            return cvec + win.astype(jnp.int32)
        lax.fori_loop(0, _K, mrg, jnp.zeros((16,), jnp.int32))

        for j in range(_KP // 16):
            sl = pl.ds(j * 16, 16)
            g = sgi[sl]
            box = jnp.minimum(g // _C, _QP - 1)
            lv[sl] = g - box * _C
            bx = box * 4
            t63 = (plsc.load_gather(cv, [bx])
                   + plsc.load_gather(cv, [bx + 2])) * 0.5
            t64 = (plsc.load_gather(cv, [bx + 1])
                   + plsc.load_gather(cv, [bx + 3])) * 0.5
            a = (t63 + t64) * 0.5
            mv[sl] = (a > 0.5).astype(jnp.int32)

        pltpu.sync_copy(stk, s_hbm.at[pl.ds(b * _KP, _KP)])
        pltpu.sync_copy(lv, l_hbm.at[pl.ds(b * _KP, _KP)])
        pltpu.sync_copy(mv, m_hbm.at[pl.ds(b * _KP, _KP)])


def kernel(pred_logits, pred_masks, target_sizes):
    B, Q, C = pred_logits.shape
    prob = jax.nn.sigmoid(pred_logits).reshape(B, Q * C)
    probp = jnp.pad(prob, ((0, 0), (0, _NP - _N)))
    probp = jax.lax.bitcast_convert_type(probp, jnp.int32).reshape(B * _NP)
    centers = pred_masks[:, :, 63:65, 63:65]
    centers = centers.astype(jnp.float16).astype(jnp.float32).reshape(B, Q, 4)
    cent = jnp.pad(centers, ((0, 0), (0, _QP - Q), (0, 0))).reshape(B * _QP * 4)

    mesh = plsc.VectorSubcoreMesh(core_axis_name="c", subcore_axis_name="s")
    s, l, mb = pl.kernel(
        _sc_body,
        mesh=mesh,
        compiler_params=pltpu.CompilerParams(needs_layout_passes=False),
        out_type=[
            jax.ShapeDtypeStruct((B * _KP,), jnp.int32),
            jax.ShapeDtypeStruct((B * _KP,), jnp.int32),
            jax.ShapeDtypeStruct((B * _KP,), jnp.int32),
        ],
        scratch_types=[
            pltpu.VMEM((_SH,), jnp.int32),
            pltpu.VMEM((_QP * 4,), jnp.float32),
            pltpu.VMEM((_CAP,), jnp.int32),
            pltpu.VMEM((_CAP,), jnp.int32),
            pltpu.VMEM((4096,), jnp.int32),
            pltpu.VMEM((_KP,), jnp.int32),
            pltpu.VMEM((_KP,), jnp.int32),
            pltpu.VMEM((_KP,), jnp.int32),
            pltpu.VMEM((_KP,), jnp.int32),
            pltpu.VMEM((2 * _NW * _KP,), jnp.int32),
            pltpu.VMEM((2 * _NW * _KP,), jnp.int32),
            pltpu.VMEM_SHARED((2 * _NW * _KP,), jnp.int32),
            pltpu.VMEM_SHARED((2 * _NW * _KP,), jnp.int32),
        ],
    )(probp, cent)
    s = s.reshape(B, _KP)
    l = l.reshape(B, _KP)
    mb = mb.reshape(B, _KP)
    scores = jax.lax.bitcast_convert_type(s[:, :_K], jnp.float32)
    labels = l[:, :_K]
    masks = mb[:, :_K].astype(bool).reshape(B, _K, 1, 1)
    return masks, scores, labels


# speedup vs baseline: 1.0325x; 1.0325x over previous
import jax
import jax.numpy as jnp
from jax import lax
from jax.experimental import pallas as pl
from jax.experimental.pallas import tpu as pltpu
from jax.experimental.pallas import tpu_sc as plsc

_K = 100
_KP = 128
_C = 91
_N = 27300
_NW = 8
_NP = 27392
_SH = _NP // _NW
_SNV = _SH // 16
_QP = 304
_CAP = _SH + 16
_BIG = 1 << 28


def _sc_body(prob_hbm, cent_hbm, s_hbm, l_hbm, m_hbm,
             pv, cv, keys, gidx, hist, stk, sgi, lv, mv, mkf, mgf,
             skeys_sh, sgidx_sh):
    cid = lax.axis_index("c")
    sid = lax.axis_index("s")
    img = sid // _NW
    b = img * 2 + cid
    shard = sid % _NW
    is_lead = shard == 0

    pltpu.sync_copy(prob_hbm.at[pl.ds(b * _NP + shard * _SH, _SH)], pv)

    @pl.when(is_lead)
    def _():
        pltpu.sync_copy(cent_hbm.at[pl.ds(b * _QP * 4, _QP * 4)], cv)

    lane = lax.broadcasted_iota(jnp.int32, (16,), 0)
    ones = jnp.ones((16,), jnp.int32)
    zeros16 = jnp.zeros((16,), jnp.int32)
    big16 = jnp.full((16,), _BIG, jnp.int32)

    def clr(j, _):
        hist[pl.ds(j * 16, 16)] = zeros16
        return 0

    def suffix_scan(acc0, nbins):
        def scan(t, carry):
            acc, bsel, nab = carry
            bin_ = nbins - 1 - t
            tsum = jnp.sum(hist[pl.ds(bin_ * 16, 16)])
            acc2 = acc + tsum
            found = (acc < _K) & (acc2 >= _K)
            bsel = jnp.where(found, bin_, bsel)
            nab = jnp.where(found, acc, nab)
            return acc2, bsel, nab
        return lax.fori_loop(
            0, nbins, scan, (acc0, jnp.int32(0), jnp.int32(0)))

    lax.fori_loop(0, 64, clr, 0)

    def ha(i, _):
        k1 = pv[pl.ds(i * 16, 16)]
        plsc.addupdate_scatter(hist, [(k1 >> 24) * 16 + lane], ones)
        return 0
    lax.fori_loop(0, _SNV, ha, 0)
    _, b1, nab1 = suffix_scan(jnp.int32(0), 64)

    lax.fori_loop(0, 256, clr, 0)

    def hb(i, _):
        k1 = pv[pl.ds(i * 16, 16)]
        plsc.addupdate_scatter(
            hist, [((k1 >> 16) & 0xFF) * 16 + lane], ones,
            mask=(k1 >> 24) == b1)
        return 0
    lax.fori_loop(0, _SNV, hb, 0)
    _, b2, _ = suffix_scan(nab1, 256)
    t16 = b1 * 256 + b2

    gbase = shard * _SH

    def cp(i, off):
        k1 = pv[pl.ds(i * 16, 16)]
        m = (k1 >> 16) >= t16
        plsc.store_compressed(keys.at[pl.ds(off, 16)], k1, mask=m)
        plsc.store_compressed(gidx.at[pl.ds(off, 16)],
                              gbase + i * 16 + lane, mask=m)
        return off + plsc.all_reduce_population_count(m)[0]
    s_cnt = lax.fori_loop(0, _SNV, cp, jnp.int32(0))

    plsc.store_scatter(keys, [s_cnt + lane], zeros16)
    plsc.store_scatter(gidx, [s_cnt + lane], big16)
    nv = (s_cnt + 15) // 16

    for j in range(_KP // 16):
        sl = pl.ds(j * 16, 16)
        stk[sl] = zeros16
        sgi[sl] = big16

    lane0 = lane == 0
    neg16 = jnp.full((16,), -1, jnp.int32)

    def emit(k, mkey, midx):
        k16 = jnp.broadcast_to(k, (16,))
        plsc.store_scatter(stk, [k16], jnp.broadcast_to(mkey, (16,)),
                           mask=lane0)
        plsc.store_scatter(sgi, [k16], jnp.broadcast_to(midx, (16,)),
                           mask=lane0)

    def mn_sweep(mkey):
        def mn(j, vi):
            kv = keys[pl.ds(j * 16, 16)]
            return jnp.minimum(
                vi, jnp.where(kv == mkey, gidx[pl.ds(j * 16, 16)],
                              jnp.int32(_BIG)))
        vi = lax.fori_loop(0, nv, mn, big16)
        return jnp.min(vi)

    def mark_sweep(mkey, midx):
        def mark(j, _):
            kv = keys[pl.ds(j * 16, 16)]
            hit = (kv == mkey) & (gidx[pl.ds(j * 16, 16)] == midx)
            keys[pl.ds(j * 16, 16)] = jnp.where(hit, jnp.int32(-1), kv)
            return 0
        lax.fori_loop(0, nv, mark, 0)

    def build_summary(j, sm):
        mj = jnp.max(keys[pl.ds(j * 16, 16)])
        return jnp.where(lane == j, mj, sm)

    @pl.when(nv <= 16)
    def _():
        summary0 = lax.fori_loop(0, nv, build_summary, neg16)

        def sel(k, summary):
            mkey = jnp.max(summary)
            hits = summary == mkey
            nhit = plsc.all_reduce_population_count(hits)[0]
            jstar = plsc.all_reduce_ffs(hits)[0]

            def fast(_):
                sl = pl.ds(jstar * 16, 16)
                kv = keys[sl]
                gv = gidx[sl]
                midx = jnp.min(jnp.where(kv == mkey, gv, jnp.int32(_BIG)))
                kv2 = jnp.where((kv == mkey) & (gv == midx),
                                jnp.int32(-1), kv)
                keys[sl] = kv2
                sm2 = jnp.where(lane == jstar, jnp.max(kv2), summary)
                return midx, sm2

            def slow(_):
                midx = mn_sweep(mkey)
                mark_sweep(mkey, midx)
                sm2 = lax.fori_loop(0, nv, build_summary, neg16)
                return midx, sm2

            midx, sm2 = lax.cond(nhit == 1, fast, slow, 0)
            emit(k, mkey, midx)
            return sm2
        lax.fori_loop(0, _K, sel, summary0)

    @pl.when(nv > 16)
    def _():
        def sel(k, _):
            def mx(j, vm):
                return jnp.maximum(vm, keys[pl.ds(j * 16, 16)])
            vm = lax.fori_loop(0, nv, mx, neg16)
            mkey = jnp.max(vm)
            midx = mn_sweep(mkey)
            mark_sweep(mkey, midx)
            emit(k, mkey, midx)
            return 0
        lax.fori_loop(0, _K, sel, 0)

    slot = img * _NW + shard
    pltpu.sync_copy(stk, skeys_sh.at[pl.ds(slot * _KP, _KP)])
    pltpu.sync_copy(sgi, sgidx_sh.at[pl.ds(slot * _KP, _KP)])
    plsc.subcore_barrier()

    @pl.when(is_lead)
    def _():
        nwk = _NW * _KP
        pltpu.sync_copy(skeys_sh.at[pl.ds(img * nwk, nwk)],
                        mkf.at[pl.ds(0, nwk)])
        pltpu.sync_copy(sgidx_sh.at[pl.ds(img * nwk, nwk)],
                        mgf.at[pl.ds(0, nwk)])
        def padr(j, _):
            mkf[pl.ds(nwk + j * 16, 16)] = zeros16
            mgf[pl.ds(nwk + j * 16, 16)] = big16
            return 0
        lax.fori_loop(0, nwk // 16, padr, 0)

        lane0 = lane == 0

        def mrg(k, cvec):
            hk = plsc.load_gather(mkf, [lane * _KP + cvec])
            hg = plsc.load_gather(mgf, [lane * _KP + cvec])
            bk = jnp.max(hk)
            bg = jnp.min(jnp.where(hk == bk, hg, jnp.int32(_BIG)))
            win = (hk == bk) & (hg == bg)
            k16 = jnp.broadcast_to(k, (16,))
            plsc.store_scatter(stk, [k16], jnp.broadcast_to(bk, (16,)),
                               mask=lane0)
            plsc.store_scatter(sgi, [k16], jnp.broadcast_to(bg, (16,)),
                               mask=lane0)
            return cvec + win.astype(jnp.int32)
        lax.fori_loop(0, _K, mrg, jnp.zeros((16,), jnp.int32))

        for j in range(_KP // 16):
            sl = pl.ds(j * 16, 16)
            g = sgi[sl]
            box = jnp.minimum(g // _C, _QP - 1)
            lv[sl] = g - box * _C
            bx = box * 4
            t63 = (plsc.load_gather(cv, [bx])
                   + plsc.load_gather(cv, [bx + 2])) * 0.5
            t64 = (plsc.load_gather(cv, [bx + 1])
                   + plsc.load_gather(cv, [bx + 3])) * 0.5
            a = (t63 + t64) * 0.5
            mv[sl] = (a > 0.5).astype(jnp.int32)

        pltpu.sync_copy(stk, s_hbm.at[pl.ds(b * _KP, _KP)])
        pltpu.sync_copy(lv, l_hbm.at[pl.ds(b * _KP, _KP)])
        pltpu.sync_copy(mv, m_hbm.at[pl.ds(b * _KP, _KP)])


def kernel(pred_logits, pred_masks, target_sizes):
    B, Q, C = pred_logits.shape
    prob = jax.nn.sigmoid(pred_logits).reshape(B, Q * C)
    probp = jnp.pad(prob, ((0, 0), (0, _NP - _N)))
    probp = jax.lax.bitcast_convert_type(probp, jnp.int32).reshape(B * _NP)
    centers = pred_masks[:, :, 63:65, 63:65]
    centers = centers.astype(jnp.float16).astype(jnp.float32).reshape(B, Q, 4)
    cent = jnp.pad(centers, ((0, 0), (0, _QP - Q), (0, 0))).reshape(B * _QP * 4)

    mesh = plsc.VectorSubcoreMesh(core_axis_name="c", subcore_axis_name="s")
    s, l, mb = pl.kernel(
        _sc_body,
        mesh=mesh,
        compiler_params=pltpu.CompilerParams(needs_layout_passes=False),
        out_type=[
            jax.ShapeDtypeStruct((B * _KP,), jnp.int32),
            jax.ShapeDtypeStruct((B * _KP,), jnp.int32),
            jax.ShapeDtypeStruct((B * _KP,), jnp.int32),
        ],
        scratch_types=[
            pltpu.VMEM((_SH,), jnp.int32),
            pltpu.VMEM((_QP * 4,), jnp.float32),
            pltpu.VMEM((_CAP,), jnp.int32),
            pltpu.VMEM((_CAP,), jnp.int32),
            pltpu.VMEM((4096,), jnp.int32),
            pltpu.VMEM((_KP,), jnp.int32),
            pltpu.VMEM((_KP,), jnp.int32),
            pltpu.VMEM((_KP,), jnp.int32),
            pltpu.VMEM((_KP,), jnp.int32),
            pltpu.VMEM((2 * _NW * _KP,), jnp.int32),
            pltpu.VMEM((2 * _NW * _KP,), jnp.int32),
            pltpu.VMEM_SHARED((2 * _NW * _KP,), jnp.int32),
            pltpu.VMEM_SHARED((2 * _NW * _KP,), jnp.int32),
        ],
    )(probp, cent)
    s = s.reshape(B, _KP)
    l = l.reshape(B, _KP)
    mb = mb.reshape(B, _KP)
    scores = jax.lax.bitcast_convert_type(s[:, :_K], jnp.float32)
    labels = l[:, :_K]
    masks = mb[:, :_K].astype(bool).reshape(B, _K, 1, 1)
    return masks, scores, labels


# speedup vs baseline: 1.1182x; 1.0830x over previous
import jax
import jax.numpy as jnp
from jax import lax
from jax.experimental import pallas as pl
from jax.experimental.pallas import tpu as pltpu
from jax.experimental.pallas import tpu_sc as plsc

_K = 100
_KP = 128
_C = 91
_N = 27300
_NW = 8
_NP = 27392
_SH = _NP // _NW
_SNV = _SH // 16
_QP = 304
_CAP = _SH + 16
_BIG = 1 << 28


def _sc_body(prob_hbm, cent_hbm, s_hbm, l_hbm, m_hbm,
             pv, cv, keys, gidx, hist, stk, sgi, lv, mv, mkf, mgf,
             skeys_sh, sgidx_sh):
    cid = lax.axis_index("c")
    sid = lax.axis_index("s")
    img = sid // _NW
    b = img * 2 + cid
    shard = sid % _NW
    is_lead = shard == 0

    pltpu.sync_copy(prob_hbm.at[pl.ds(b * _NP + shard * _SH, _SH)], pv)

    @pl.when(is_lead)
    def _():
        pltpu.sync_copy(cent_hbm.at[pl.ds(b * _QP * 4, _QP * 4)], cv)

    lane = lax.broadcasted_iota(jnp.int32, (16,), 0)
    ones = jnp.ones((16,), jnp.int32)
    zeros16 = jnp.zeros((16,), jnp.int32)
    big16 = jnp.full((16,), _BIG, jnp.int32)

    def clrloop(n):
        @plsc.parallel_loop(0, n, unroll=4)
        def _(j):
            hist[pl.ds(j * 16, 16)] = zeros16

    def suffix_scan(acc0, nbins):
        def scan(t, carry):
            acc, bsel, nab = carry
            bin_ = nbins - 1 - t
            tsum = jnp.sum(hist[pl.ds(bin_ * 16, 16)])
            acc2 = acc + tsum
            found = (acc < _K) & (acc2 >= _K)
            bsel = jnp.where(found, bin_, bsel)
            nab = jnp.where(found, acc, nab)
            return acc2, bsel, nab
        return lax.fori_loop(
            0, nbins, scan, (acc0, jnp.int32(0), jnp.int32(0)))

    clrloop(64)

    @plsc.parallel_loop(0, _SNV, unroll=4)
    def _ha(i):
        k1 = pv[pl.ds(i * 16, 16)]
        plsc.addupdate_scatter(hist, [(k1 >> 24) * 16 + lane], ones)
    _, b1, nab1 = suffix_scan(jnp.int32(0), 64)

    clrloop(256)

    @plsc.parallel_loop(0, _SNV, unroll=4)
    def _hb(i):
        k1 = pv[pl.ds(i * 16, 16)]
        plsc.addupdate_scatter(
            hist, [((k1 >> 16) & 0xFF) * 16 + lane], ones,
            mask=(k1 >> 24) == b1)
    _, b2, _ = suffix_scan(nab1, 256)
    t16 = b1 * 256 + b2

    gbase = shard * _SH

    def cp(i, off):
        k1 = pv[pl.ds(i * 16, 16)]
        m = (k1 >> 16) >= t16
        plsc.store_compressed(keys.at[pl.ds(off, 16)], k1, mask=m)
        plsc.store_compressed(gidx.at[pl.ds(off, 16)],
                              gbase + i * 16 + lane, mask=m)
        return off + plsc.all_reduce_population_count(m)[0]
    s_cnt = lax.fori_loop(0, _SNV, cp, jnp.int32(0))

    plsc.store_scatter(keys, [s_cnt + lane], zeros16)
    plsc.store_scatter(gidx, [s_cnt + lane], big16)
    nv = (s_cnt + 15) // 16

    for j in range(_KP // 16):
        sl = pl.ds(j * 16, 16)
        stk[sl] = zeros16
        sgi[sl] = big16

    lane0 = lane == 0
    neg16 = jnp.full((16,), -1, jnp.int32)

    def emit(k, mkey, midx):
        k16 = jnp.broadcast_to(k, (16,))
        plsc.store_scatter(stk, [k16], jnp.broadcast_to(mkey, (16,)),
                           mask=lane0)
        plsc.store_scatter(sgi, [k16], jnp.broadcast_to(midx, (16,)),
                           mask=lane0)

    def mn_sweep(mkey):
        def mn(j, vi):
            kv = keys[pl.ds(j * 16, 16)]
            return jnp.minimum(
                vi, jnp.where(kv == mkey, gidx[pl.ds(j * 16, 16)],
                              jnp.int32(_BIG)))
        vi = lax.fori_loop(0, nv, mn, big16)
        return jnp.min(vi)

    def mark_sweep(mkey, midx):
        def mark(j, _):
            kv = keys[pl.ds(j * 16, 16)]
            hit = (kv == mkey) & (gidx[pl.ds(j * 16, 16)] == midx)
            keys[pl.ds(j * 16, 16)] = jnp.where(hit, jnp.int32(-1), kv)
            return 0
        lax.fori_loop(0, nv, mark, 0)

    def build_summary(j, sm):
        mj = jnp.max(keys[pl.ds(j * 16, 16)])
        return jnp.where(lane == j, mj, sm)

    @pl.when(nv <= 16)
    def _():
        summary0 = lax.fori_loop(0, nv, build_summary, neg16)

        def sel(k, summary):
            mkey = jnp.max(summary)
            hits = summary == mkey
            nhit = plsc.all_reduce_population_count(hits)[0]
            jstar = plsc.all_reduce_ffs(hits)[0]

            def fast(_):
                sl = pl.ds(jstar * 16, 16)
                kv = keys[sl]
                gv = gidx[sl]
                midx = jnp.min(jnp.where(kv == mkey, gv, jnp.int32(_BIG)))
                kv2 = jnp.where((kv == mkey) & (gv == midx),
                                jnp.int32(-1), kv)
                keys[sl] = kv2
                sm2 = jnp.where(lane == jstar, jnp.max(kv2), summary)
                return midx, sm2

            def slow(_):
                midx = mn_sweep(mkey)
                mark_sweep(mkey, midx)
                sm2 = lax.fori_loop(0, nv, build_summary, neg16)
                return midx, sm2

            midx, sm2 = lax.cond(nhit == 1, fast, slow, 0)
            emit(k, mkey, midx)
            return sm2
        lax.fori_loop(0, _K, sel, summary0)

    @pl.when(nv > 16)
    def _():
        def sel(k, _):
            def mx(j, vm):
                return jnp.maximum(vm, keys[pl.ds(j * 16, 16)])
            vm = lax.fori_loop(0, nv, mx, neg16)
            mkey = jnp.max(vm)
            midx = mn_sweep(mkey)
            mark_sweep(mkey, midx)
            emit(k, mkey, midx)
            return 0
        lax.fori_loop(0, _K, sel, 0)

    slot = img * _NW + shard
    pltpu.sync_copy(stk, skeys_sh.at[pl.ds(slot * _KP, _KP)])
    pltpu.sync_copy(sgi, sgidx_sh.at[pl.ds(slot * _KP, _KP)])
    plsc.subcore_barrier()

    @pl.when(is_lead)
    def _():
        nwk = _NW * _KP
        pltpu.sync_copy(skeys_sh.at[pl.ds(img * nwk, nwk)],
                        mkf.at[pl.ds(0, nwk)])
        pltpu.sync_copy(sgidx_sh.at[pl.ds(img * nwk, nwk)],
                        mgf.at[pl.ds(0, nwk)])
        def padr(j, _):
            mkf[pl.ds(nwk + j * 16, 16)] = zeros16
            mgf[pl.ds(nwk + j * 16, 16)] = big16
            return 0
        lax.fori_loop(0, nwk // 16, padr, 0)

        lane0 = lane == 0

        def mrg(k, cvec):
            hk = plsc.load_gather(mkf, [lane * _KP + cvec])
            hg = plsc.load_gather(mgf, [lane * _KP + cvec])
            bk = jnp.max(hk)
            bg = jnp.min(jnp.where(hk == bk, hg, jnp.int32(_BIG)))
            win = (hk == bk) & (hg == bg)
            k16 = jnp.broadcast_to(k, (16,))
            plsc.store_scatter(stk, [k16], jnp.broadcast_to(bk, (16,)),
                               mask=lane0)
            plsc.store_scatter(sgi, [k16], jnp.broadcast_to(bg, (16,)),
                               mask=lane0)
            return cvec + win.astype(jnp.int32)
        lax.fori_loop(0, _K, mrg, jnp.zeros((16,), jnp.int32))

        for j in range(_KP // 16):
            sl = pl.ds(j * 16, 16)
            g = sgi[sl]
            box = jnp.minimum(g // _C, _QP - 1)
            lv[sl] = g - box * _C
            bx = box * 4
            t63 = (plsc.load_gather(cv, [bx])
                   + plsc.load_gather(cv, [bx + 2])) * 0.5
            t64 = (plsc.load_gather(cv, [bx + 1])
                   + plsc.load_gather(cv, [bx + 3])) * 0.5
            a = (t63 + t64) * 0.5
            mv[sl] = (a > 0.5).astype(jnp.int32)

        pltpu.sync_copy(stk, s_hbm.at[pl.ds(b * _KP, _KP)])
        pltpu.sync_copy(lv, l_hbm.at[pl.ds(b * _KP, _KP)])
        pltpu.sync_copy(mv, m_hbm.at[pl.ds(b * _KP, _KP)])


def kernel(pred_logits, pred_masks, target_sizes):
    B, Q, C = pred_logits.shape
    prob = jax.nn.sigmoid(pred_logits).reshape(B, Q * C)
    probp = jnp.pad(prob, ((0, 0), (0, _NP - _N)))
    probp = jax.lax.bitcast_convert_type(probp, jnp.int32).reshape(B * _NP)
    centers = pred_masks[:, :, 63:65, 63:65]
    centers = centers.astype(jnp.float16).astype(jnp.float32).reshape(B, Q, 4)
    cent = jnp.pad(centers, ((0, 0), (0, _QP - Q), (0, 0))).reshape(B * _QP * 4)

    mesh = plsc.VectorSubcoreMesh(core_axis_name="c", subcore_axis_name="s")
    s, l, mb = pl.kernel(
        _sc_body,
        mesh=mesh,
        compiler_params=pltpu.CompilerParams(needs_layout_passes=False),
        out_type=[
            jax.ShapeDtypeStruct((B * _KP,), jnp.int32),
            jax.ShapeDtypeStruct((B * _KP,), jnp.int32),
            jax.ShapeDtypeStruct((B * _KP,), jnp.int32),
        ],
        scratch_types=[
            pltpu.VMEM((_SH,), jnp.int32),
            pltpu.VMEM((_QP * 4,), jnp.float32),
            pltpu.VMEM((_CAP,), jnp.int32),
            pltpu.VMEM((_CAP,), jnp.int32),
            pltpu.VMEM((4096,), jnp.int32),
            pltpu.VMEM((_KP,), jnp.int32),
            pltpu.VMEM((_KP,), jnp.int32),
            pltpu.VMEM((_KP,), jnp.int32),
            pltpu.VMEM((_KP,), jnp.int32),
            pltpu.VMEM((2 * _NW * _KP,), jnp.int32),
            pltpu.VMEM((2 * _NW * _KP,), jnp.int32),
            pltpu.VMEM_SHARED((2 * _NW * _KP,), jnp.int32),
            pltpu.VMEM_SHARED((2 * _NW * _KP,), jnp.int32),
        ],
    )(probp, cent)
    s = s.reshape(B, _KP)
    l = l.reshape(B, _KP)
    mb = mb.reshape(B, _KP)
    scores = jax.lax.bitcast_convert_type(s[:, :_K], jnp.float32)
    labels = l[:, :_K]
    masks = mb[:, :_K].astype(bool).reshape(B, _K, 1, 1)
    return masks, scores, labels


# speedup vs baseline: 1.1582x; 1.0358x over previous
import jax
import jax.numpy as jnp
from jax import lax
from jax.experimental import pallas as pl
from jax.experimental.pallas import tpu as pltpu
from jax.experimental.pallas import tpu_sc as plsc

_K = 100
_KP = 128
_C = 91
_N = 27300
_NW = 8
_NP = 27392
_SH = _NP // _NW
_SNV = _SH // 16
_QP = 304
_CAP = _SH + 16
_BIG = 1 << 28


def _sc_body(prob_hbm, cent_hbm, s_hbm, l_hbm, m_hbm,
             pv, cv, keys, gidx, hist, stk, sgi, lv, mv, mkf, mgf,
             skeys_sh, sgidx_sh):
    cid = lax.axis_index("c")
    sid = lax.axis_index("s")
    img = sid // _NW
    b = img * 2 + cid
    shard = sid % _NW
    is_lead = shard == 0

    pltpu.sync_copy(prob_hbm.at[pl.ds(b * _NP + shard * _SH, _SH)], pv)

    @pl.when(is_lead)
    def _():
        pltpu.sync_copy(cent_hbm.at[pl.ds(b * _QP * 4, _QP * 4)], cv)

    lane = lax.broadcasted_iota(jnp.int32, (16,), 0)
    ones = jnp.ones((16,), jnp.int32)
    zeros16 = jnp.zeros((16,), jnp.int32)
    big16 = jnp.full((16,), _BIG, jnp.int32)

    def clrloop(n):
        @plsc.parallel_loop(0, n, unroll=4)
        def _(j):
            hist[pl.ds(j * 16, 16)] = zeros16

    def suffix_scan(acc0, nbins):
        def scan(t, carry):
            acc, bsel, nab = carry
            bin_ = nbins - 1 - t
            tsum = jnp.sum(hist[pl.ds(bin_ * 16, 16)])
            acc2 = acc + tsum
            found = (acc < _K) & (acc2 >= _K)
            bsel = jnp.where(found, bin_, bsel)
            nab = jnp.where(found, acc, nab)
            return acc2, bsel, nab
        return lax.fori_loop(
            0, nbins, scan, (acc0, jnp.int32(0), jnp.int32(0)))

    clrloop(64)

    @plsc.parallel_loop(0, _SNV, unroll=4)
    def _ha(i):
        k1 = pv[pl.ds(i * 16, 16)]
        plsc.addupdate_scatter(hist, [(k1 >> 24) * 16 + lane], ones)
    _, b1, nab1 = suffix_scan(jnp.int32(0), 64)

    clrloop(256)

    @plsc.parallel_loop(0, _SNV, unroll=4)
    def _hb(i):
        k1 = pv[pl.ds(i * 16, 16)]
        plsc.addupdate_scatter(
            hist, [((k1 >> 16) & 0xFF) * 16 + lane], ones,
            mask=(k1 >> 24) == b1)
    _, b2, _ = suffix_scan(nab1, 256)
    t16 = b1 * 256 + b2

    gbase = shard * _SH

    @plsc.parallel_loop(0, _SNV, unroll=2, carry=jnp.int32(0))
    def s_cnt(i, off):
        k1 = pv[pl.ds(i * 16, 16)]
        m = (k1 >> 16) >= t16
        plsc.store_compressed(keys.at[pl.ds(off, 16)], k1, mask=m)
        plsc.store_compressed(gidx.at[pl.ds(off, 16)],
                              gbase + i * 16 + lane, mask=m)
        return off + plsc.all_reduce_population_count(m)[0]

    plsc.store_scatter(keys, [s_cnt + lane], zeros16)
    plsc.store_scatter(gidx, [s_cnt + lane], big16)
    nv = (s_cnt + 15) // 16

    for j in range(_KP // 16):
        sl = pl.ds(j * 16, 16)
        stk[sl] = zeros16
        sgi[sl] = big16

    lane0 = lane == 0
    neg16 = jnp.full((16,), -1, jnp.int32)

    def emit(k, mkey, midx):
        k16 = jnp.broadcast_to(k, (16,))
        plsc.store_scatter(stk, [k16], jnp.broadcast_to(mkey, (16,)),
                           mask=lane0)
        plsc.store_scatter(sgi, [k16], jnp.broadcast_to(midx, (16,)),
                           mask=lane0)

    def mn_sweep(mkey):
        def mn(j, vi):
            kv = keys[pl.ds(j * 16, 16)]
            return jnp.minimum(
                vi, jnp.where(kv == mkey, gidx[pl.ds(j * 16, 16)],
                              jnp.int32(_BIG)))
        vi = lax.fori_loop(0, nv, mn, big16)
        return jnp.min(vi)

    def mark_sweep(mkey, midx):
        def mark(j, _):
            kv = keys[pl.ds(j * 16, 16)]
            hit = (kv == mkey) & (gidx[pl.ds(j * 16, 16)] == midx)
            keys[pl.ds(j * 16, 16)] = jnp.where(hit, jnp.int32(-1), kv)
            return 0
        lax.fori_loop(0, nv, mark, 0)

    def build_summary(j, sm):
        mj = jnp.max(keys[pl.ds(j * 16, 16)])
        return jnp.where(lane == j, mj, sm)

    @pl.when(nv <= 16)
    def _():
        summary0 = lax.fori_loop(0, nv, build_summary, neg16)

        def sel(k, summary):
            mkey = jnp.max(summary)
            hits = summary == mkey
            nhit = plsc.all_reduce_population_count(hits)[0]
            jstar = plsc.all_reduce_ffs(hits)[0]

            def fast(_):
                sl = pl.ds(jstar * 16, 16)
                kv = keys[sl]
                gv = gidx[sl]
                midx = jnp.min(jnp.where(kv == mkey, gv, jnp.int32(_BIG)))
                kv2 = jnp.where((kv == mkey) & (gv == midx),
                                jnp.int32(-1), kv)
                keys[sl] = kv2
                sm2 = jnp.where(lane == jstar, jnp.max(kv2), summary)
                return midx, sm2

            def slow(_):
                midx = mn_sweep(mkey)
                mark_sweep(mkey, midx)
                sm2 = lax.fori_loop(0, nv, build_summary, neg16)
                return midx, sm2

            midx, sm2 = lax.cond(nhit == 1, fast, slow, 0)
            emit(k, mkey, midx)
            return sm2
        lax.fori_loop(0, _K, sel, summary0)

    @pl.when(nv > 16)
    def _():
        def sel(k, _):
            def mx(j, vm):
                return jnp.maximum(vm, keys[pl.ds(j * 16, 16)])
            vm = lax.fori_loop(0, nv, mx, neg16)
            mkey = jnp.max(vm)
            midx = mn_sweep(mkey)
            mark_sweep(mkey, midx)
            emit(k, mkey, midx)
            return 0
        lax.fori_loop(0, _K, sel, 0)

    slot = img * _NW + shard
    pltpu.sync_copy(stk, skeys_sh.at[pl.ds(slot * _KP, _KP)])
    pltpu.sync_copy(sgi, sgidx_sh.at[pl.ds(slot * _KP, _KP)])
    plsc.subcore_barrier()

    @pl.when(is_lead)
    def _():
        nwk = _NW * _KP
        pltpu.sync_copy(skeys_sh.at[pl.ds(img * nwk, nwk)],
                        mkf.at[pl.ds(0, nwk)])
        pltpu.sync_copy(sgidx_sh.at[pl.ds(img * nwk, nwk)],
                        mgf.at[pl.ds(0, nwk)])
        def padr(j, _):
            mkf[pl.ds(nwk + j * 16, 16)] = zeros16
            mgf[pl.ds(nwk + j * 16, 16)] = big16
            return 0
        lax.fori_loop(0, nwk // 16, padr, 0)

        lane0 = lane == 0

        def mrg(k, cvec):
            hk = plsc.load_gather(mkf, [lane * _KP + cvec])
            hg = plsc.load_gather(mgf, [lane * _KP + cvec])
            bk = jnp.max(hk)
            bg = jnp.min(jnp.where(hk == bk, hg, jnp.int32(_BIG)))
            win = (hk == bk) & (hg == bg)
            k16 = jnp.broadcast_to(k, (16,))
            plsc.store_scatter(stk, [k16], jnp.broadcast_to(bk, (16,)),
                               mask=lane0)
            plsc.store_scatter(sgi, [k16], jnp.broadcast_to(bg, (16,)),
                               mask=lane0)
            return cvec + win.astype(jnp.int32)
        lax.fori_loop(0, _K, mrg, jnp.zeros((16,), jnp.int32))

        for j in range(_KP // 16):
            sl = pl.ds(j * 16, 16)
            g = sgi[sl]
            box = jnp.minimum(g // _C, _QP - 1)
            lv[sl] = g - box * _C
            bx = box * 4
            t63 = (plsc.load_gather(cv, [bx])
                   + plsc.load_gather(cv, [bx + 2])) * 0.5
            t64 = (plsc.load_gather(cv, [bx + 1])
                   + plsc.load_gather(cv, [bx + 3])) * 0.5
            a = (t63 + t64) * 0.5
            mv[sl] = (a > 0.5).astype(jnp.int32)

        pltpu.sync_copy(stk, s_hbm.at[pl.ds(b * _KP, _KP)])
        pltpu.sync_copy(lv, l_hbm.at[pl.ds(b * _KP, _KP)])
        pltpu.sync_copy(mv, m_hbm.at[pl.ds(b * _KP, _KP)])


def kernel(pred_logits, pred_masks, target_sizes):
    B, Q, C = pred_logits.shape
    prob = jax.nn.sigmoid(pred_logits).reshape(B, Q * C)
    probp = jnp.pad(prob, ((0, 0), (0, _NP - _N)))
    probp = jax.lax.bitcast_convert_type(probp, jnp.int32).reshape(B * _NP)
    centers = pred_masks[:, :, 63:65, 63:65]
    centers = centers.astype(jnp.float16).astype(jnp.float32).reshape(B, Q, 4)
    cent = jnp.pad(centers, ((0, 0), (0, _QP - Q), (0, 0))).reshape(B * _QP * 4)

    mesh = plsc.VectorSubcoreMesh(core_axis_name="c", subcore_axis_name="s")
    s, l, mb = pl.kernel(
        _sc_body,
        mesh=mesh,
        compiler_params=pltpu.CompilerParams(needs_layout_passes=False),
        out_type=[
            jax.ShapeDtypeStruct((B * _KP,), jnp.int32),
            jax.ShapeDtypeStruct((B * _KP,), jnp.int32),
            jax.ShapeDtypeStruct((B * _KP,), jnp.int32),
        ],
        scratch_types=[
            pltpu.VMEM((_SH,), jnp.int32),
            pltpu.VMEM((_QP * 4,), jnp.float32),
            pltpu.VMEM((_CAP,), jnp.int32),
            pltpu.VMEM((_CAP,), jnp.int32),
            pltpu.VMEM((4096,), jnp.int32),
            pltpu.VMEM((_KP,), jnp.int32),
            pltpu.VMEM((_KP,), jnp.int32),
            pltpu.VMEM((_KP,), jnp.int32),
            pltpu.VMEM((_KP,), jnp.int32),
            pltpu.VMEM((2 * _NW * _KP,), jnp.int32),
            pltpu.VMEM((2 * _NW * _KP,), jnp.int32),
            pltpu.VMEM_SHARED((2 * _NW * _KP,), jnp.int32),
            pltpu.VMEM_SHARED((2 * _NW * _KP,), jnp.int32),
        ],
    )(probp, cent)
    s = s.reshape(B, _KP)
    l = l.reshape(B, _KP)
    mb = mb.reshape(B, _KP)
    scores = jax.lax.bitcast_convert_type(s[:, :_K], jnp.float32)
    labels = l[:, :_K]
    masks = mb[:, :_K].astype(bool).reshape(B, _K, 1, 1)
    return masks, scores, labels
